# Initial kernel scaffold; baseline (speedup 1.0000x reference)
import jax
import jax.numpy as jnp
from jax.experimental import pallas as pl


def kernel(x_u, x_c, x_o, ea_vc, ea_ov, ea_oc, ei_vc, ei_ov, ei_oc, params):
    raise NotImplementedError("write your pallas kernel here")


# speedup vs baseline: 2.0643x; 2.0643x over previous
import functools

import jax
import jax.numpy as jnp
import numpy as _np
from jax import lax
from jax.experimental import pallas as pl
from jax.experimental.pallas import tpu as pltpu
from jax.experimental.pallas import tpu_sc as plsc

F = 64
N = 10000
EVC = 320000
NWORK = 32
EPW = EVC // NWORK
CH = 80
NCHUNK = EPW // CH
RPT = 624
TAIL0 = 16 * RPT


def _mmT(x, w):
    return lax.dot_general(x, w, (((1,), (1,)), ((), ())),
                           preferred_element_type=jnp.float32)


def _ln(x, g, b, eps=1e-5):
    m = jnp.mean(x, axis=-1, keepdims=True)
    v = jnp.var(x, axis=-1, keepdims=True)
    return (x - m) / jnp.sqrt(v + eps) * g + b


def _embed_body(x_ref, lnp_ref, w1_ref, b1_ref, w2_ref, b2_ref, o_ref):
    x = x_ref[...]
    h = _ln(x, lnp_ref[0], lnp_ref[1])
    h = jax.nn.relu(_mmT(h, w1_ref[...]) + b1_ref[...])
    h = jax.nn.relu(_mmT(h, w2_ref[...]) + b2_ref[...])
    o_ref[...] = h


def _embed(x, p, blk):
    n, w = x.shape
    grid = n // blk
    lnp = jnp.stack([p['ln_g'], p['ln_b']])
    return pl.pallas_call(
        _embed_body,
        grid=(grid,),
        in_specs=[
            pl.BlockSpec((blk, w), lambda i: (i, 0)),
            pl.BlockSpec((2, w), lambda i: (0, 0)),
            pl.BlockSpec((F, w), lambda i: (0, 0)),
            pl.BlockSpec((1, F), lambda i: (0, 0)),
            pl.BlockSpec((F, F), lambda i: (0, 0)),
            pl.BlockSpec((1, F), lambda i: (0, 0)),
        ],
        out_specs=pl.BlockSpec((blk, F), lambda i: (i, 0)),
        out_shape=jax.ShapeDtypeStruct((n, F), jnp.float32),
    )(x, lnp, p['W1'], p['b1'][None, :], p['W2'], p['b2'][None, :])


def _edge_ln_body(ea_ref, eln_ref, o_ref):
    o_ref[...] = _ln(ea_ref[...], eln_ref[0], eln_ref[1])


def _edge_ln(ea, eg, eb, blk):
    n = ea.shape[0]
    eln = jnp.stack([eg, eb])
    return pl.pallas_call(
        _edge_ln_body,
        grid=(n // blk,),
        in_specs=[pl.BlockSpec((blk, 2), lambda i: (i, 0)),
                  pl.BlockSpec((2, 2), lambda i: (0, 0))],
        out_specs=pl.BlockSpec((blk, 2), lambda i: (i, 0)),
        out_shape=jax.ShapeDtypeStruct((n, 2), jnp.float32),
    )(ea, eln)


def _dense_conv_body(x_ref, ea_ref, onode_ref, mats_ref, vecs_ref, ev_ref,
                     eln_ref, o_ref):
    x = x_ref[...]
    ean = _ln(ea_ref[...], eln_ref[0], eln_ref[1])
    t = _mmT(x, mats_ref[0]) + vecs_ref[0]
    t = t + ean[:, 0:1] * ev_ref[0] + ean[:, 1:2] * ev_ref[1]
    t = t + _mmT(onode_ref[...], mats_ref[1])
    m = jax.nn.relu(_ln(t, vecs_ref[1], vecs_ref[2]))
    s = _mmT(m, mats_ref[2]) + vecs_ref[3]
    z = _ln(s, vecs_ref[4], vecs_ref[5])
    h = jax.nn.relu(_mmT(x, mats_ref[3]) + _mmT(z, mats_ref[4]) + vecs_ref[6])
    o_ref[...] = _mmT(h, mats_ref[5]) + vecs_ref[7]


def _dense_conv(x, ea, onode, q, dq, eg, eb, blk):
    n = x.shape[0]
    mats = jnp.stack([q['Wl'], q['Wr'], q['Wf'],
                      dq['W1'][:, :F], dq['W1'][:, F:], dq['W2']])
    vecs = jnp.stack([q['bl'], q['g'], q['bln'], q['bf'],
                      _LN_G, _LN_B, dq['b1'], dq['b2']])
    ev = jnp.stack([q['We'][:, 0], q['We'][:, 1]])
    eln = jnp.stack([eg, eb])
    return pl.pallas_call(
        _dense_conv_body,
        grid=(n // blk,),
        in_specs=[
            pl.BlockSpec((blk, F), lambda i: (i, 0)),
            pl.BlockSpec((blk, 2), lambda i: (i, 0)),
            pl.BlockSpec((1, F), lambda i: (0, 0)),
            pl.BlockSpec((6, F, F), lambda i: (0, 0, 0)),
            pl.BlockSpec((8, F), lambda i: (0, 0)),
            pl.BlockSpec((2, F), lambda i: (0, 0)),
            pl.BlockSpec((2, 2), lambda i: (0, 0)),
        ],
        out_specs=pl.BlockSpec((blk, F), lambda i: (i, 0)),
        out_shape=jax.ShapeDtypeStruct((n, F), jnp.float32),
    )(x, ea, onode, mats, vecs, ev, eln)


def _reduce_conv_body(x_ref, ea_ref, onode_ref, mats_ref, vecs_ref, ev_ref,
                      eln_ref, s_ref, onew_ref, *, nrows):
    i = pl.program_id(0)
    ng = pl.num_programs(0)
    x = x_ref[...]
    ean = _ln(ea_ref[...], eln_ref[0], eln_ref[1])
    t = _mmT(onode_ref[...], mats_ref[0]) + vecs_ref[0]
    t = t + ean[:, 0:1] * ev_ref[0] + ean[:, 1:2] * ev_ref[1]
    t = t + _mmT(x, mats_ref[1])
    m = jax.nn.relu(_ln(t, vecs_ref[1], vecs_ref[2]))
    part = jnp.sum(m, axis=0, keepdims=True)

    @pl.when(i == 0)
    def _():
        s_ref[...] = part

    @pl.when(i > 0)
    def _():
        s_ref[...] = s_ref[...] + part

    @pl.when(i == ng - 1)
    def _():
        s = _mmT(s_ref[...], mats_ref[2]) + nrows * vecs_ref[3]
        z = _ln(s, vecs_ref[4], vecs_ref[5])
        onode = onode_ref[...]
        h = jax.nn.relu(_mmT(onode, mats_ref[3]) + _mmT(z, mats_ref[4])
                        + vecs_ref[6])
        onew_ref[...] = _mmT(h, mats_ref[5]) + vecs_ref[7]


def _reduce_conv(x, ea, onode, q, dq, eg, eb, blk):
    n = x.shape[0]
    mats = jnp.stack([q['Wl'], q['Wr'], q['Wf'],
                      dq['W1'][:, :F], dq['W1'][:, F:], dq['W2']])
    vecs = jnp.stack([q['bl'], q['g'], q['bln'], q['bf'],
                      _LN_G, _LN_B, dq['b1'], dq['b2']])
    ev = jnp.stack([q['We'][:, 0], q['We'][:, 1]])
    eln = jnp.stack([eg, eb])
    _, onew = pl.pallas_call(
        functools.partial(_reduce_conv_body, nrows=float(n)),
        grid=(n // blk,),
        in_specs=[
            pl.BlockSpec((blk, F), lambda i: (i, 0)),
            pl.BlockSpec((blk, 2), lambda i: (i, 0)),
            pl.BlockSpec((1, F), lambda i: (0, 0)),
            pl.BlockSpec((6, F, F), lambda i: (0, 0, 0)),
            pl.BlockSpec((8, F), lambda i: (0, 0)),
            pl.BlockSpec((2, F), lambda i: (0, 0)),
            pl.BlockSpec((2, 2), lambda i: (0, 0)),
        ],
        out_specs=[pl.BlockSpec((1, F), lambda i: (0, 0)),
                   pl.BlockSpec((1, F), lambda i: (0, 0))],
        out_shape=[jax.ShapeDtypeStruct((1, F), jnp.float32),
                   jax.ShapeDtypeStruct((1, F), jnp.float32)],
    )(x, ea, onode, mats, vecs, ev, eln)
    return onew


def _pre_conv_body(xr_ref, xl_ref, wl_ref, bl_ref, wr_ref, a_ref, b_ref):
    a_ref[...] = _mmT(xr_ref[...], wl_ref[...]) + bl_ref[...]
    b_ref[...] = _mmT(xl_ref[...], wr_ref[...])


def _pre_conv(x_right, x_left, q, blk):
    n = x_right.shape[0]
    return pl.pallas_call(
        _pre_conv_body,
        grid=(n // blk,),
        in_specs=[
            pl.BlockSpec((blk, F), lambda i: (i, 0)),
            pl.BlockSpec((blk, F), lambda i: (i, 0)),
            pl.BlockSpec((F, F), lambda i: (0, 0)),
            pl.BlockSpec((1, F), lambda i: (0, 0)),
            pl.BlockSpec((F, F), lambda i: (0, 0)),
        ],
        out_specs=[pl.BlockSpec((blk, F), lambda i: (i, 0)),
                   pl.BlockSpec((blk, F), lambda i: (i, 0))],
        out_shape=[jax.ShapeDtypeStruct((n, F), jnp.float32),
                   jax.ShapeDtypeStruct((n, F), jnp.float32)],
    )(x_right, x_left, q['Wl'], q['bl'][None, :], q['Wr'])


def _post_conv_body(x_ref, sp_ref, degp_ref, mats_ref, vecs_ref, o_ref):
    x = x_ref[...]
    s_sum = sp_ref[0] + sp_ref[1]
    deg = degp_ref[0, :, 0] + degp_ref[1, :, 0]
    s = _mmT(s_sum, mats_ref[0]) + deg[:, None] * vecs_ref[0]
    z = _ln(s, vecs_ref[1], vecs_ref[2])
    h = jax.nn.relu(_mmT(x, mats_ref[1]) + _mmT(z, mats_ref[2]) + vecs_ref[3])
    o_ref[...] = _mmT(h, mats_ref[3]) + vecs_ref[4]


def _post_conv(x, sp, degp, q, dq, blk):
    n = x.shape[0]
    mats = jnp.stack([q['Wf'], dq['W1'][:, :F], dq['W1'][:, F:], dq['W2']])
    vecs = jnp.stack([q['bf'], _LN_G, _LN_B, dq['b1'], dq['b2']])
    return pl.pallas_call(
        _post_conv_body,
        grid=(n // blk,),
        in_specs=[
            pl.BlockSpec((blk, F), lambda i: (i, 0)),
            pl.BlockSpec((2, blk, F), lambda i: (0, i, 0)),
            pl.BlockSpec((2, blk, 16), lambda i: (0, i, 0)),
            pl.BlockSpec((4, F, F), lambda i: (0, 0, 0)),
            pl.BlockSpec((5, F), lambda i: (0, 0)),
        ],
        out_specs=pl.BlockSpec((blk, F), lambda i: (i, 0)),
        out_shape=jax.ShapeDtypeStruct((n, F), jnp.float32),
    )(x, sp, degp, mats, vecs)


def _head_body(x_ref, w1_ref, b1_ref, w2_ref, o_ref):
    h = jax.nn.relu(_mmT(x_ref[...], w1_ref[...]) + b1_ref[...])
    o_ref[...] = jax.nn.sigmoid(_mmT(h, w2_ref[...]))


def _head(x, w1, b1, w2, blk):
    n = x.shape[0]
    return pl.pallas_call(
        _head_body,
        grid=(n // blk,),
        in_specs=[
            pl.BlockSpec((blk, F), lambda i: (i, 0)),
            pl.BlockSpec((F, F), lambda i: (0, 0)),
            pl.BlockSpec((1, F), lambda i: (0, 0)),
            pl.BlockSpec((1, F), lambda i: (0, 0)),
        ],
        out_specs=pl.BlockSpec((blk, 1), lambda i: (i, 0)),
        out_shape=jax.ShapeDtypeStruct((n, 1), jnp.float32),
    )(x, w1, b1, w2)


_GDN = lax.GatherDimensionNumbers(offset_dims=(), collapsed_slice_dims=(0,),
                                  start_index_map=(0,))


def _hsum16(t):
    lanes = lax.iota(jnp.int32, 16)
    for st in (8, 4, 2, 1):
        idx = (lanes ^ st).reshape(16, 1)
        t = t + lax.gather(t, idx, _GDN, (1,),
                           mode=lax.GatherScatterMode.PROMISE_IN_BOUNDS)
    return t


def _rsqrt16(v):
    i = lax.bitcast_convert_type(v, jnp.int32)
    y = lax.bitcast_convert_type(jnp.int32(0x5F3759DF) - (i >> 1), jnp.float32)
    for _ in range(3):
        y = y * (1.5 - 0.5 * v * y * y)
    return y


def _sc_edge_conv_body(a_hbm, b_hbm, src_hbm, dst_hbm, ea_hbm, lnp_hbm,
                       zeros_hbm, out_hbm, prm_v, srcv, dstv, eav, ra, rb,
                       mb, acc, sem_a, sem_b):
    cid = lax.axis_index("c")
    sid = lax.axis_index("s")
    wid = sid * 2 + cid
    base = wid * EPW

    pltpu.sync_copy(lnp_hbm, prm_v)
    r0 = pl.multiple_of(sid * RPT, 8)
    pltpu.sync_copy(zeros_hbm.at[pl.ds(r0, RPT)], acc.at[pl.ds(r0, RPT)])

    @pl.when(sid == 15)
    def _():
        pltpu.sync_copy(zeros_hbm.at[pl.ds(TAIL0, 16)],
                        acc.at[pl.ds(TAIL0, 16)])

    plsc.subcore_barrier()

    def chunk(j, carry):
        off = pl.multiple_of(base + j * CH, 8)
        pltpu.sync_copy(src_hbm.at[pl.ds(off, CH)], srcv)
        pltpu.sync_copy(dst_hbm.at[pl.ds(off, CH)], dstv)
        pltpu.sync_copy(ea_hbm.at[pl.ds(off * 2, CH * 2)],
                        eav.at[pl.ds(0, CH * 2)])
        cp_a = pltpu.async_copy(a_hbm.at[dstv], ra, sem_a)
        cp_b = pltpu.async_copy(b_hbm.at[srcv], rb, sem_b)
        cp_a.wait()
        cp_b.wait()

        def edge(e, carry2):
            h = [ra[e, pl.ds(16 * k, 16)] + rb[e, pl.ds(16 * k, 16)]
                 for k in range(4)]
            eap = eav[pl.ds(2 * e, 16)]
            ea0 = eap[0]
            ea1 = eap[1]
            h = [h[k] + ea0 * prm_v[0, pl.ds(16 * k, 16)]
                 + ea1 * prm_v[1, pl.ds(16 * k, 16)] for k in range(4)]
            t = (h[0] + h[1]) + (h[2] + h[3])
            tsq = (h[0] * h[0] + h[1] * h[1]) + (h[2] * h[2] + h[3] * h[3])
            mean = _hsum16(t) * (1.0 / 64.0)
            var = jnp.maximum(_hsum16(tsq) * (1.0 / 64.0) - mean * mean, 0.0)
            rstd = _rsqrt16(var + 1e-5)
            for k in range(4):
                mk = (h[k] - mean) * rstd
                mk = mk * prm_v[2, pl.ds(16 * k, 16)] \
                    + prm_v[3, pl.ds(16 * k, 16)]
                mb[e, pl.ds(16 * k, 16)] = jnp.maximum(mk, 0.0)
            return carry2

        lax.fori_loop(0, CH, edge, 0, unroll=False)
        pltpu.sync_copy(mb, acc.at[dstv], add=True)
        return carry

    lax.fori_loop(0, NCHUNK, chunk, 0, unroll=False)
    plsc.subcore_barrier()
    pltpu.sync_copy(acc.at[pl.ds(r0, RPT)],
                    out_hbm.at[cid].at[pl.ds(r0, RPT)])

    @pl.when(sid == 15)
    def _():
        pltpu.sync_copy(acc.at[pl.ds(TAIL0, 16)],
                        out_hbm.at[cid].at[pl.ds(TAIL0, 16)])


_sc_edge_conv_call = pl.kernel(
    _sc_edge_conv_body,
    mesh=plsc.VectorSubcoreMesh(core_axis_name="c", subcore_axis_name="s"),
    compiler_params=pltpu.CompilerParams(use_tc_tiling_on_sc=False),
    out_type=jax.ShapeDtypeStruct((2, N, F), jnp.float32),
    scratch_types=[
        pltpu.VMEM((4, F), jnp.float32),
        pltpu.VMEM((CH,), jnp.int32),
        pltpu.VMEM((CH,), jnp.int32),
        pltpu.VMEM((CH * 2 + 16,), jnp.float32),
        pltpu.VMEM((CH, F), jnp.float32),
        pltpu.VMEM((CH, F), jnp.float32),
        pltpu.VMEM((CH, F), jnp.float32),
        pltpu.VMEM_SHARED((N, F), jnp.float32),
        pltpu.SemaphoreType.DMA,
        pltpu.SemaphoreType.DMA,
    ],
)


def _sc_edge_conv(a, b, src, dst, ean, q, zeros64):
    lnp = jnp.stack([q['We'][:, 0], q['We'][:, 1], q['g'], q['bln']])
    return _sc_edge_conv_call(a, b, src, dst, ean.ravel(), lnp, zeros64)


def _sc_degree_body(idx_hbm, zeros_hbm, out_hbm, idxv, ones_v, acc, sem):
    cid = lax.axis_index("c")
    sid = lax.axis_index("s")
    wid = sid * 2 + cid
    base = wid * EPW

    def fill(i, carry):
        ones_v[i, pl.ds(0, 16)] = jnp.full((16,), 1.0, jnp.float32)
        return carry

    lax.fori_loop(0, CH, fill, 0, unroll=False)
    r0 = pl.multiple_of(sid * RPT, 8)
    pltpu.sync_copy(zeros_hbm.at[pl.ds(r0, RPT)], acc.at[pl.ds(r0, RPT)])

    @pl.when(sid == 15)
    def _():
        pltpu.sync_copy(zeros_hbm.at[pl.ds(TAIL0, 16)],
                        acc.at[pl.ds(TAIL0, 16)])

    plsc.subcore_barrier()

    def chunk(j, carry):
        off = pl.multiple_of(base + j * CH, 8)
        pltpu.sync_copy(idx_hbm.at[pl.ds(off, CH)], idxv)
        pltpu.sync_copy(ones_v, acc.at[idxv], add=True)
        return carry

    lax.fori_loop(0, NCHUNK, chunk, 0, unroll=False)
    plsc.subcore_barrier()
    pltpu.sync_copy(acc.at[pl.ds(r0, RPT)],
                    out_hbm.at[cid].at[pl.ds(r0, RPT)])

    @pl.when(sid == 15)
    def _():
        pltpu.sync_copy(acc.at[pl.ds(TAIL0, 16)],
                        out_hbm.at[cid].at[pl.ds(TAIL0, 16)])


_sc_degree = pl.kernel(
    _sc_degree_body,
    mesh=plsc.VectorSubcoreMesh(core_axis_name="c", subcore_axis_name="s"),
    compiler_params=pltpu.CompilerParams(use_tc_tiling_on_sc=False),
    out_type=jax.ShapeDtypeStruct((2, N, 16), jnp.float32),
    scratch_types=[
        pltpu.VMEM((CH,), jnp.int32),
        pltpu.VMEM((CH, 16), jnp.float32),
        pltpu.VMEM_SHARED((N, 16), jnp.float32),
        pltpu.SemaphoreType.DMA,
    ],
)


_LN_G = None
_LN_B = None


def kernel(x_u, x_c, x_o, ea_vc, ea_ov, ea_oc, ei_vc, ei_ov, ei_oc, params):
    global _LN_G, _LN_B
    p = params
    _LN_G, _LN_B = p['ln_g'], p['ln_b']
    eg, eb = p['edge_ln_g'], p['edge_ln_b']

    src_v = ei_vc[0]
    dst_c = ei_vc[1]
    zeros64 = jnp.zeros((N, F), jnp.float32)
    zeros16 = jnp.zeros((N, 16), jnp.float32)

    u = _embed(x_u, p['ne0'], 2000)
    c = _embed(x_c, p['ne1'], 2000)
    o = _embed(x_o, p['ne2'], 1)
    ean_vc = _edge_ln(ea_vc, eg, eb, 20000)

    degp_c = _sc_degree(dst_c, zeros16)
    degp_v = _sc_degree(src_v, zeros16)

    blk = 2000
    for l in range(2):
        o = _reduce_conv(u, ea_ov, o, p['conv%d_u_obj' % l],
                         p['emb%d_obj' % l], eg, eb, blk)
        c = _dense_conv(c, ea_oc, o, p['conv%d_obj_con' % l],
                        p['emb%d_con' % l], eg, eb, blk)
        q = p['conv%d_u_con' % l]
        a, b = _pre_conv(c, u, q, blk)
        sp = _sc_edge_conv(a, b, src_v, dst_c, ean_vc, q, zeros64)
        c = _post_conv(c, sp, degp_c, q, p['emb%d_con' % l], blk)
        o = _reduce_conv(c, ea_oc, o, p['conv%d_con_obj' % l],
                         p['emb%d_obj' % l], eg, eb, blk)
        u = _dense_conv(u, ea_ov, o, p['conv%d_obj_u' % l],
                        p['emb%d_u' % l], eg, eb, blk)
        q = p['conv%d_con_u' % l]
        a, b = _pre_conv(u, c, q, blk)
        sp = _sc_edge_conv(a, b, dst_c, src_v, ean_vc, q, zeros64)
        u = _post_conv(u, sp, degp_v, q, p['emb%d_u' % l], blk)

    return _head(u, p['out_W1'], p['out_b1'][None, :], p['out_W2'], 2000)


# speedup vs baseline: 2.2761x; 1.1026x over previous
import functools

import jax
import jax.numpy as jnp
import numpy as _np
from jax import lax
from jax.experimental import pallas as pl
from jax.experimental.pallas import tpu as pltpu
from jax.experimental.pallas import tpu_sc as plsc

F = 64
N = 10000
EVC = 320000
NWORK = 32
EPW = EVC // NWORK
CH = 80
NCHUNK = EPW // CH
RPT = 624
TAIL0 = 16 * RPT


def _mmT(x, w):
    return lax.dot_general(x, w, (((1,), (1,)), ((), ())),
                           preferred_element_type=jnp.float32)


def _ln(x, g, b, eps=1e-5):
    m = jnp.mean(x, axis=-1, keepdims=True)
    v = jnp.var(x, axis=-1, keepdims=True)
    return (x - m) / jnp.sqrt(v + eps) * g + b


def _embed_body(x_ref, lnp_ref, w1_ref, b1_ref, w2_ref, b2_ref, o_ref):
    x = x_ref[...]
    h = _ln(x, lnp_ref[0], lnp_ref[1])
    h = jax.nn.relu(_mmT(h, w1_ref[...]) + b1_ref[...])
    h = jax.nn.relu(_mmT(h, w2_ref[...]) + b2_ref[...])
    o_ref[...] = h


def _embed(x, p, blk):
    n, w = x.shape
    grid = n // blk
    lnp = jnp.stack([p['ln_g'], p['ln_b']])
    return pl.pallas_call(
        _embed_body,
        grid=(grid,),
        in_specs=[
            pl.BlockSpec((blk, w), lambda i: (i, 0)),
            pl.BlockSpec((2, w), lambda i: (0, 0)),
            pl.BlockSpec((F, w), lambda i: (0, 0)),
            pl.BlockSpec((1, F), lambda i: (0, 0)),
            pl.BlockSpec((F, F), lambda i: (0, 0)),
            pl.BlockSpec((1, F), lambda i: (0, 0)),
        ],
        out_specs=pl.BlockSpec((blk, F), lambda i: (i, 0)),
        out_shape=jax.ShapeDtypeStruct((n, F), jnp.float32),
    )(x, lnp, p['W1'], p['b1'][None, :], p['W2'], p['b2'][None, :])


def _edge_ln_body(ea_ref, eln_ref, o_ref):
    o_ref[...] = _ln(ea_ref[...], eln_ref[0], eln_ref[1])


def _edge_ln(ea, eg, eb, blk):
    n = ea.shape[0]
    eln = jnp.stack([eg, eb])
    return pl.pallas_call(
        _edge_ln_body,
        grid=(n // blk,),
        in_specs=[pl.BlockSpec((blk, 2), lambda i: (i, 0)),
                  pl.BlockSpec((2, 2), lambda i: (0, 0))],
        out_specs=pl.BlockSpec((blk, 2), lambda i: (i, 0)),
        out_shape=jax.ShapeDtypeStruct((n, 2), jnp.float32),
    )(ea, eln)


def _dense_conv_body(x_ref, ea_ref, onode_ref, mats_ref, vecs_ref, ev_ref,
                     eln_ref, o_ref):
    x = x_ref[...]
    ean = _ln(ea_ref[...], eln_ref[0], eln_ref[1])
    t = _mmT(x, mats_ref[0]) + vecs_ref[0]
    t = t + ean[:, 0:1] * ev_ref[0] + ean[:, 1:2] * ev_ref[1]
    t = t + _mmT(onode_ref[...], mats_ref[1])
    m = jax.nn.relu(_ln(t, vecs_ref[1], vecs_ref[2]))
    s = _mmT(m, mats_ref[2]) + vecs_ref[3]
    z = _ln(s, vecs_ref[4], vecs_ref[5])
    h = jax.nn.relu(_mmT(x, mats_ref[3]) + _mmT(z, mats_ref[4]) + vecs_ref[6])
    o_ref[...] = _mmT(h, mats_ref[5]) + vecs_ref[7]


def _dense_conv(x, ea, onode, q, dq, eg, eb, blk):
    n = x.shape[0]
    mats = jnp.stack([q['Wl'], q['Wr'], q['Wf'],
                      dq['W1'][:, :F], dq['W1'][:, F:], dq['W2']])
    vecs = jnp.stack([q['bl'], q['g'], q['bln'], q['bf'],
                      _LN_G, _LN_B, dq['b1'], dq['b2']])
    ev = jnp.stack([q['We'][:, 0], q['We'][:, 1]])
    eln = jnp.stack([eg, eb])
    return pl.pallas_call(
        _dense_conv_body,
        grid=(n // blk,),
        in_specs=[
            pl.BlockSpec((blk, F), lambda i: (i, 0)),
            pl.BlockSpec((blk, 2), lambda i: (i, 0)),
            pl.BlockSpec((1, F), lambda i: (0, 0)),
            pl.BlockSpec((6, F, F), lambda i: (0, 0, 0)),
            pl.BlockSpec((8, F), lambda i: (0, 0)),
            pl.BlockSpec((2, F), lambda i: (0, 0)),
            pl.BlockSpec((2, 2), lambda i: (0, 0)),
        ],
        out_specs=pl.BlockSpec((blk, F), lambda i: (i, 0)),
        out_shape=jax.ShapeDtypeStruct((n, F), jnp.float32),
    )(x, ea, onode, mats, vecs, ev, eln)


def _reduce_conv_body(x_ref, ea_ref, onode_ref, mats_ref, vecs_ref, ev_ref,
                      eln_ref, s_ref, onew_ref, *, nrows):
    i = pl.program_id(0)
    ng = pl.num_programs(0)
    x = x_ref[...]
    ean = _ln(ea_ref[...], eln_ref[0], eln_ref[1])
    t = _mmT(onode_ref[...], mats_ref[0]) + vecs_ref[0]
    t = t + ean[:, 0:1] * ev_ref[0] + ean[:, 1:2] * ev_ref[1]
    t = t + _mmT(x, mats_ref[1])
    m = jax.nn.relu(_ln(t, vecs_ref[1], vecs_ref[2]))
    part = jnp.sum(m, axis=0, keepdims=True)

    @pl.when(i == 0)
    def _():
        s_ref[...] = part

    @pl.when(i > 0)
    def _():
        s_ref[...] = s_ref[...] + part

    @pl.when(i == ng - 1)
    def _():
        s = _mmT(s_ref[...], mats_ref[2]) + nrows * vecs_ref[3]
        z = _ln(s, vecs_ref[4], vecs_ref[5])
        onode = onode_ref[...]
        h = jax.nn.relu(_mmT(onode, mats_ref[3]) + _mmT(z, mats_ref[4])
                        + vecs_ref[6])
        onew_ref[...] = _mmT(h, mats_ref[5]) + vecs_ref[7]


def _reduce_conv(x, ea, onode, q, dq, eg, eb, blk):
    n = x.shape[0]
    mats = jnp.stack([q['Wl'], q['Wr'], q['Wf'],
                      dq['W1'][:, :F], dq['W1'][:, F:], dq['W2']])
    vecs = jnp.stack([q['bl'], q['g'], q['bln'], q['bf'],
                      _LN_G, _LN_B, dq['b1'], dq['b2']])
    ev = jnp.stack([q['We'][:, 0], q['We'][:, 1]])
    eln = jnp.stack([eg, eb])
    _, onew = pl.pallas_call(
        functools.partial(_reduce_conv_body, nrows=float(n)),
        grid=(n // blk,),
        in_specs=[
            pl.BlockSpec((blk, F), lambda i: (i, 0)),
            pl.BlockSpec((blk, 2), lambda i: (i, 0)),
            pl.BlockSpec((1, F), lambda i: (0, 0)),
            pl.BlockSpec((6, F, F), lambda i: (0, 0, 0)),
            pl.BlockSpec((8, F), lambda i: (0, 0)),
            pl.BlockSpec((2, F), lambda i: (0, 0)),
            pl.BlockSpec((2, 2), lambda i: (0, 0)),
        ],
        out_specs=[pl.BlockSpec((1, F), lambda i: (0, 0)),
                   pl.BlockSpec((1, F), lambda i: (0, 0))],
        out_shape=[jax.ShapeDtypeStruct((1, F), jnp.float32),
                   jax.ShapeDtypeStruct((1, F), jnp.float32)],
    )(x, ea, onode, mats, vecs, ev, eln)
    return onew


def _pre_conv_body(xr_ref, xl_ref, wl_ref, bl_ref, wr_ref, a_ref, b_ref):
    a_ref[...] = _mmT(xr_ref[...], wl_ref[...]) + bl_ref[...]
    b_ref[...] = _mmT(xl_ref[...], wr_ref[...])


def _pre_conv(x_right, x_left, q, blk):
    n = x_right.shape[0]
    return pl.pallas_call(
        _pre_conv_body,
        grid=(n // blk,),
        in_specs=[
            pl.BlockSpec((blk, F), lambda i: (i, 0)),
            pl.BlockSpec((blk, F), lambda i: (i, 0)),
            pl.BlockSpec((F, F), lambda i: (0, 0)),
            pl.BlockSpec((1, F), lambda i: (0, 0)),
            pl.BlockSpec((F, F), lambda i: (0, 0)),
        ],
        out_specs=[pl.BlockSpec((blk, F), lambda i: (i, 0)),
                   pl.BlockSpec((blk, F), lambda i: (i, 0))],
        out_shape=[jax.ShapeDtypeStruct((n, F), jnp.float32),
                   jax.ShapeDtypeStruct((n, F), jnp.float32)],
    )(x_right, x_left, q['Wl'], q['bl'][None, :], q['Wr'])


def _post_conv_body(x_ref, sp_ref, degp_ref, mats_ref, vecs_ref, o_ref):
    x = x_ref[...]
    s_sum = sp_ref[0] + sp_ref[1]
    deg = degp_ref[0, :, 0] + degp_ref[1, :, 0]
    s = _mmT(s_sum, mats_ref[0]) + deg[:, None] * vecs_ref[0]
    z = _ln(s, vecs_ref[1], vecs_ref[2])
    h = jax.nn.relu(_mmT(x, mats_ref[1]) + _mmT(z, mats_ref[2]) + vecs_ref[3])
    o_ref[...] = _mmT(h, mats_ref[3]) + vecs_ref[4]


def _post_conv(x, sp, degp, q, dq, blk):
    n = x.shape[0]
    mats = jnp.stack([q['Wf'], dq['W1'][:, :F], dq['W1'][:, F:], dq['W2']])
    vecs = jnp.stack([q['bf'], _LN_G, _LN_B, dq['b1'], dq['b2']])
    return pl.pallas_call(
        _post_conv_body,
        grid=(n // blk,),
        in_specs=[
            pl.BlockSpec((blk, F), lambda i: (i, 0)),
            pl.BlockSpec((2, blk, F), lambda i: (0, i, 0)),
            pl.BlockSpec((2, blk, 16), lambda i: (0, i, 0)),
            pl.BlockSpec((4, F, F), lambda i: (0, 0, 0)),
            pl.BlockSpec((5, F), lambda i: (0, 0)),
        ],
        out_specs=pl.BlockSpec((blk, F), lambda i: (i, 0)),
        out_shape=jax.ShapeDtypeStruct((n, F), jnp.float32),
    )(x, sp, degp, mats, vecs)


def _head_body(x_ref, w1_ref, b1_ref, w2_ref, o_ref):
    h = jax.nn.relu(_mmT(x_ref[...], w1_ref[...]) + b1_ref[...])
    o_ref[...] = jax.nn.sigmoid(_mmT(h, w2_ref[...]))


def _head(x, w1, b1, w2, blk):
    n = x.shape[0]
    return pl.pallas_call(
        _head_body,
        grid=(n // blk,),
        in_specs=[
            pl.BlockSpec((blk, F), lambda i: (i, 0)),
            pl.BlockSpec((F, F), lambda i: (0, 0)),
            pl.BlockSpec((1, F), lambda i: (0, 0)),
            pl.BlockSpec((1, F), lambda i: (0, 0)),
        ],
        out_specs=pl.BlockSpec((blk, 1), lambda i: (i, 0)),
        out_shape=jax.ShapeDtypeStruct((n, 1), jnp.float32),
    )(x, w1, b1, w2)


_GDN = lax.GatherDimensionNumbers(offset_dims=(), collapsed_slice_dims=(0,),
                                  start_index_map=(0,))


def _hsum16(t):
    lanes = lax.iota(jnp.int32, 16)
    for st in (8, 4, 2, 1):
        idx = (lanes ^ st).reshape(16, 1)
        t = t + lax.gather(t, idx, _GDN, (1,),
                           mode=lax.GatherScatterMode.PROMISE_IN_BOUNDS)
    return t


def _rsqrt16(v):
    i = lax.bitcast_convert_type(v, jnp.int32)
    y = lax.bitcast_convert_type(jnp.int32(0x5F3759DF) - (i >> 1), jnp.float32)
    for _ in range(3):
        y = y * (1.5 - 0.5 * v * y * y)
    return y


def _sc_edge_conv_body(a_hbm, b_hbm, src_hbm, dst_hbm, ea_hbm, lnp_hbm,
                       zeros_hbm, out_hbm, prm_v,
                       srcv0, dstv0, eav0, ra0, rb0, sem_a0, sem_b0,
                       srcv1, dstv1, eav1, ra1, rb1, sem_a1, sem_b1,
                       mb, acc):
    cid = lax.axis_index("c")
    sid = lax.axis_index("s")
    wid = sid * 2 + cid
    base = wid * EPW
    set0 = (srcv0, dstv0, eav0, ra0, rb0, sem_a0, sem_b0)
    set1 = (srcv1, dstv1, eav1, ra1, rb1, sem_a1, sem_b1)

    pltpu.sync_copy(lnp_hbm, prm_v)
    r0 = pl.multiple_of(sid * RPT, 8)
    pltpu.sync_copy(zeros_hbm.at[pl.ds(r0, RPT)], acc.at[pl.ds(r0, RPT)])

    @pl.when(sid == 15)
    def _():
        pltpu.sync_copy(zeros_hbm.at[pl.ds(TAIL0, 16)],
                        acc.at[pl.ds(TAIL0, 16)])

    plsc.subcore_barrier()

    def issue(j, bufs):
        srcv, dstv, eav, ra, rb, sem_a, sem_b = bufs
        off = pl.multiple_of(base + j * CH, 8)
        pltpu.sync_copy(src_hbm.at[pl.ds(off, CH)], srcv)
        pltpu.sync_copy(dst_hbm.at[pl.ds(off, CH)], dstv)
        pltpu.sync_copy(ea_hbm.at[pl.ds(off * 2, CH * 2)],
                        eav.at[pl.ds(0, CH * 2)])
        pltpu.async_copy(a_hbm.at[dstv], ra, sem_a)
        pltpu.async_copy(b_hbm.at[srcv], rb, sem_b)

    def wait_compute(bufs):
        srcv, dstv, eav, ra, rb, sem_a, sem_b = bufs
        pltpu.make_async_copy(a_hbm.at[dstv], ra, sem_a).wait()
        pltpu.make_async_copy(b_hbm.at[srcv], rb, sem_b).wait()

        def edge(e, carry2):
            h = [ra[e, pl.ds(16 * k, 16)] + rb[e, pl.ds(16 * k, 16)]
                 for k in range(4)]
            eap = eav[pl.ds(2 * e, 16)]
            ea0 = eap[0]
            ea1 = eap[1]
            h = [h[k] + ea0 * prm_v[0, pl.ds(16 * k, 16)]
                 + ea1 * prm_v[1, pl.ds(16 * k, 16)] for k in range(4)]
            t = (h[0] + h[1]) + (h[2] + h[3])
            tsq = (h[0] * h[0] + h[1] * h[1]) + (h[2] * h[2] + h[3] * h[3])
            mean = _hsum16(t) * (1.0 / 64.0)
            var = jnp.maximum(_hsum16(tsq) * (1.0 / 64.0) - mean * mean, 0.0)
            rstd = _rsqrt16(var + 1e-5)
            for k in range(4):
                mk = (h[k] - mean) * rstd
                mk = mk * prm_v[2, pl.ds(16 * k, 16)] \
                    + prm_v[3, pl.ds(16 * k, 16)]
                mb[e, pl.ds(16 * k, 16)] = jnp.maximum(mk, 0.0)
            return carry2

        lax.fori_loop(0, CH, edge, 0, unroll=False)
        pltpu.sync_copy(mb, acc.at[dstv], add=True)

    issue(0, set0)

    def two_chunks(i, carry):
        j0 = 2 * i
        issue(j0 + 1, set1)
        wait_compute(set0)
        issue(j0 + 2, set0)
        wait_compute(set1)
        return carry

    lax.fori_loop(0, (NCHUNK - 1) // 2, two_chunks, 0, unroll=False)
    wait_compute(set0)
    plsc.subcore_barrier()
    pltpu.sync_copy(acc.at[pl.ds(r0, RPT)],
                    out_hbm.at[cid].at[pl.ds(r0, RPT)])

    @pl.when(sid == 15)
    def _():
        pltpu.sync_copy(acc.at[pl.ds(TAIL0, 16)],
                        out_hbm.at[cid].at[pl.ds(TAIL0, 16)])


_sc_edge_conv_call = pl.kernel(
    _sc_edge_conv_body,
    mesh=plsc.VectorSubcoreMesh(core_axis_name="c", subcore_axis_name="s"),
    compiler_params=pltpu.CompilerParams(use_tc_tiling_on_sc=False),
    out_type=jax.ShapeDtypeStruct((2, N, F), jnp.float32),
    scratch_types=[
        pltpu.VMEM((4, F), jnp.float32),
        pltpu.VMEM((CH,), jnp.int32),
        pltpu.VMEM((CH,), jnp.int32),
        pltpu.VMEM((CH * 2 + 16,), jnp.float32),
        pltpu.VMEM((CH, F), jnp.float32),
        pltpu.VMEM((CH, F), jnp.float32),
        pltpu.SemaphoreType.DMA,
        pltpu.SemaphoreType.DMA,
        pltpu.VMEM((CH,), jnp.int32),
        pltpu.VMEM((CH,), jnp.int32),
        pltpu.VMEM((CH * 2 + 16,), jnp.float32),
        pltpu.VMEM((CH, F), jnp.float32),
        pltpu.VMEM((CH, F), jnp.float32),
        pltpu.SemaphoreType.DMA,
        pltpu.SemaphoreType.DMA,
        pltpu.VMEM((CH, F), jnp.float32),
        pltpu.VMEM_SHARED((N, F), jnp.float32),
    ],
)


def _sc_edge_conv(a, b, src, dst, ean, q, zeros64):
    lnp = jnp.stack([q['We'][:, 0], q['We'][:, 1], q['g'], q['bln']])
    return _sc_edge_conv_call(a, b, src, dst, ean.ravel(), lnp, zeros64)


def _sc_degree_body(idx_hbm, zeros_hbm, out_hbm, idxv, ones_v, acc, sem):
    cid = lax.axis_index("c")
    sid = lax.axis_index("s")
    wid = sid * 2 + cid
    base = wid * EPW

    def fill(i, carry):
        ones_v[i, pl.ds(0, 16)] = jnp.full((16,), 1.0, jnp.float32)
        return carry

    lax.fori_loop(0, CH, fill, 0, unroll=False)
    r0 = pl.multiple_of(sid * RPT, 8)
    pltpu.sync_copy(zeros_hbm.at[pl.ds(r0, RPT)], acc.at[pl.ds(r0, RPT)])

    @pl.when(sid == 15)
    def _():
        pltpu.sync_copy(zeros_hbm.at[pl.ds(TAIL0, 16)],
                        acc.at[pl.ds(TAIL0, 16)])

    plsc.subcore_barrier()

    def chunk(j, carry):
        off = pl.multiple_of(base + j * CH, 8)
        pltpu.sync_copy(idx_hbm.at[pl.ds(off, CH)], idxv)
        pltpu.sync_copy(ones_v, acc.at[idxv], add=True)
        return carry

    lax.fori_loop(0, NCHUNK, chunk, 0, unroll=False)
    plsc.subcore_barrier()
    pltpu.sync_copy(acc.at[pl.ds(r0, RPT)],
                    out_hbm.at[cid].at[pl.ds(r0, RPT)])

    @pl.when(sid == 15)
    def _():
        pltpu.sync_copy(acc.at[pl.ds(TAIL0, 16)],
                        out_hbm.at[cid].at[pl.ds(TAIL0, 16)])


_sc_degree = pl.kernel(
    _sc_degree_body,
    mesh=plsc.VectorSubcoreMesh(core_axis_name="c", subcore_axis_name="s"),
    compiler_params=pltpu.CompilerParams(use_tc_tiling_on_sc=False),
    out_type=jax.ShapeDtypeStruct((2, N, 16), jnp.float32),
    scratch_types=[
        pltpu.VMEM((CH,), jnp.int32),
        pltpu.VMEM((CH, 16), jnp.float32),
        pltpu.VMEM_SHARED((N, 16), jnp.float32),
        pltpu.SemaphoreType.DMA,
    ],
)


_LN_G = None
_LN_B = None


def kernel(x_u, x_c, x_o, ea_vc, ea_ov, ea_oc, ei_vc, ei_ov, ei_oc, params):
    global _LN_G, _LN_B
    p = params
    _LN_G, _LN_B = p['ln_g'], p['ln_b']
    eg, eb = p['edge_ln_g'], p['edge_ln_b']

    src_v = ei_vc[0]
    dst_c = ei_vc[1]
    zeros64 = jnp.zeros((N, F), jnp.float32)
    zeros16 = jnp.zeros((N, 16), jnp.float32)

    u = _embed(x_u, p['ne0'], 2000)
    c = _embed(x_c, p['ne1'], 2000)
    o = _embed(x_o, p['ne2'], 1)
    ean_vc = _edge_ln(ea_vc, eg, eb, 20000)

    degp_c = _sc_degree(dst_c, zeros16)
    degp_v = _sc_degree(src_v, zeros16)

    blk = 2000
    for l in range(2):
        o = _reduce_conv(u, ea_ov, o, p['conv%d_u_obj' % l],
                         p['emb%d_obj' % l], eg, eb, blk)
        c = _dense_conv(c, ea_oc, o, p['conv%d_obj_con' % l],
                        p['emb%d_con' % l], eg, eb, blk)
        q = p['conv%d_u_con' % l]
        a, b = _pre_conv(c, u, q, blk)
        sp = _sc_edge_conv(a, b, src_v, dst_c, ean_vc, q, zeros64)
        c = _post_conv(c, sp, degp_c, q, p['emb%d_con' % l], blk)
        o = _reduce_conv(c, ea_oc, o, p['conv%d_con_obj' % l],
                         p['emb%d_obj' % l], eg, eb, blk)
        u = _dense_conv(u, ea_ov, o, p['conv%d_obj_u' % l],
                        p['emb%d_u' % l], eg, eb, blk)
        q = p['conv%d_con_u' % l]
        a, b = _pre_conv(u, c, q, blk)
        sp = _sc_edge_conv(a, b, dst_c, src_v, ean_vc, q, zeros64)
        u = _post_conv(u, sp, degp_v, q, p['emb%d_u' % l], blk)

    return _head(u, p['out_W1'], p['out_b1'][None, :], p['out_W2'], 2000)
